# baseline (device time: 247008 ns/iter reference)
import os

import jax
import jax.numpy as jnp
from jax import lax
from jax.experimental import pallas as pl
from jax.experimental.pallas import tpu as pltpu

K = 2048
M_SHARD = 1024
M_BLK = 256
F = 8192
DY_CHUNK = 512
N_DY = F // DY_CHUNK
CW = int(os.environ.get("KCW", "512"))
NC = F // CW
SUB = CW // DY_CHUNK
H = CW // 2

DO_Y = "y" not in os.environ.get("KSKIP", "")
DO_X = "x" not in os.environ.get("KSKIP", "")
DO_Z = "z" not in os.environ.get("KSKIP", "")
DO_GEMM = not os.environ.get("KNOGEMM")
DO_ADD = not os.environ.get("KNOADD")
DO_FWD = DO_X and DO_Z


def kernel(x, dy):
    my_x = lax.axis_index("x")
    my_y = lax.axis_index("y")
    my_z = lax.axis_index("z")
    q = 2 * my_z + my_x

    col_me = my_y * M_SHARD + q * M_BLK
    col_pr = (1 - my_y) * M_SHARD + q * M_BLK
    a_me = lax.dynamic_slice(x, (0, col_me), (K, M_BLK)).T
    a_pr = lax.dynamic_slice(x, (0, col_pr), (K, M_BLK)).T

    def body(a_me_ref, a_pr_ref, dy_ref, out_ref,
             dy_vmem, send_buf, recv_buf,
             dy_sems, y_ss, y_rs, x1_ss, x1_rs, z1_ss, z1_rs,
             z2_ss, z2_rs, x2_ss, x2_rs):
        my_x = lax.axis_index("x")
        my_y = lax.axis_index("y")
        my_z = lax.axis_index("z")
        row0 = (2 * my_z + my_x) * M_BLK
        row0x = (2 * my_z + (1 - my_x)) * M_BLK
        row0z = (2 * (1 - my_z) + my_x) * M_BLK

        y_dev = (my_x, 1 - my_y, my_z)
        x_dev = (1 - my_x, my_y, my_z)
        z_dev = (my_x, my_y, 1 - my_z)

        barrier = pltpu.get_barrier_semaphore()
        for dev in (y_dev, x_dev, z_dev):
            pl.semaphore_signal(barrier, inc=1, device_id=dev,
                                device_id_type=pl.DeviceIdType.MESH)
        pl.semaphore_wait(barrier, 3)

        def exch(src_rows, col_off, width, ss, rs, dev):
            return pltpu.make_async_remote_copy(
                src_ref=out_ref.at[pl.ds(src_rows, M_BLK),
                                   pl.ds(col_off, width)],
                dst_ref=out_ref.at[pl.ds(src_rows, M_BLK),
                                   pl.ds(col_off, width)],
                send_sem=ss, recv_sem=rs,
                device_id=dev, device_id_type=pl.DeviceIdType.MESH)

        def make_y(c):
            cols = pl.ds(c * CW, CW)
            return pltpu.make_async_remote_copy(
                src_ref=send_buf.at[:, cols], dst_ref=recv_buf.at[:, cols],
                send_sem=y_ss.at[c], recv_sem=y_rs.at[c],
                device_id=y_dev, device_id_type=pl.DeviceIdType.MESH)

        y_rdmas = [make_y(c) for c in range(NC)]
        x1_rdmas = [exch(row0, c * CW, CW, x1_ss.at[c], x1_rs.at[c], x_dev)
                    for c in range(NC)]
        z1_rdmas = [exch(row0, c * CW, CW, z1_ss.at[c], z1_rs.at[c], z_dev)
                    for c in range(NC)]
        z2_rdmas = [exch(row0x, c * CW, H, z2_ss.at[c], z2_rs.at[c], z_dev)
                    for c in range(NC)]
        x2_rdmas = [exch(row0z, c * CW + H, H, x2_ss.at[c], x2_rs.at[c], x_dev)
                    for c in range(NC)]

        def dy_copy(i):
            cols = pl.ds(i * DY_CHUNK, DY_CHUNK)
            return pltpu.make_async_copy(
                dy_ref.at[:, cols], dy_vmem.at[i % 2], dy_sems.at[i % 2])

        if DO_GEMM:
            copies = [dy_copy(i) for i in range(N_DY)]
            copies[0].start()
            for i in range(N_DY):
                copies[i].wait()
                cols = pl.ds(i * DY_CHUNK, DY_CHUNK)
                d = dy_vmem[i % 2, :, :]
                send_buf[:, cols] = lax.dot_general(
                    a_pr_ref[:, :], d, (((1,), (0,)), ((), ())),
                    preferred_element_type=jnp.float32)
                if DO_Y and (i + 1) % SUB == 0:
                    y_rdmas[(i + 1) // SUB - 1].start()
                if i + 1 < N_DY:
                    copies[i + 1].start()
                out_ref[pl.ds(row0, M_BLK), cols] = lax.dot_general(
                    a_me_ref[:, :], d, (((1,), (0,)), ((), ())),
                    preferred_element_type=jnp.float32)
        elif DO_Y:
            for c in range(NC):
                y_rdmas[c].start()

        for c in range(NC):
            cols = pl.ds(c * CW, CW)
            if DO_Y:
                y_rdmas[c].wait_recv()
                if DO_ADD:
                    out_ref[pl.ds(row0, M_BLK), cols] = (
                        out_ref[pl.ds(row0, M_BLK), cols] + recv_buf[:, cols])
            if DO_Z:
                z1_rdmas[c].start()
            if DO_X:
                x1_rdmas[c].start()

        if DO_X:
            for c in range(NC):
                x1_rdmas[c].wait_recv()
                if DO_FWD:
                    z2_rdmas[c].start()

        if DO_Z:
            for c in range(NC):
                z1_rdmas[c].wait_recv()
                if DO_FWD:
                    x2_rdmas[c].start()

        for c in range(NC):
            if DO_FWD:
                z2_rdmas[c].wait_recv()
                x2_rdmas[c].wait_recv()
            if DO_Y:
                y_rdmas[c].wait_send()
            if DO_X:
                x1_rdmas[c].wait_send()
            if DO_Z:
                z1_rdmas[c].wait_send()
            if DO_FWD:
                z2_rdmas[c].wait_send()
                x2_rdmas[c].wait_send()

    dma = pltpu.SemaphoreType.DMA
    return pl.pallas_call(
        body,
        out_shape=jax.ShapeDtypeStruct((M_SHARD, F), jnp.float32),
        in_specs=[
            pl.BlockSpec(memory_space=pltpu.VMEM),
            pl.BlockSpec(memory_space=pltpu.VMEM),
            pl.BlockSpec(memory_space=pl.ANY),
        ],
        out_specs=pl.BlockSpec(memory_space=pltpu.VMEM),
        scratch_shapes=[
            pltpu.VMEM((2, K, DY_CHUNK), jnp.float32),
            pltpu.VMEM((M_BLK, F), jnp.float32),
            pltpu.VMEM((M_BLK, F), jnp.float32),
            dma((2,)),
            dma((NC,)), dma((NC,)),
            dma((NC,)), dma((NC,)),
            dma((NC,)), dma((NC,)),
            dma((NC,)), dma((NC,)),
            dma((NC,)), dma((NC,)),
        ],
        compiler_params=pltpu.CompilerParams(
            collective_id=0, vmem_limit_bytes=64 * 1024 * 1024),
    )(a_me, a_pr, dy)


# device time: 236752 ns/iter; 1.0433x vs baseline; 1.0433x over previous
import os

import jax
import jax.numpy as jnp
from jax import lax
from jax.experimental import pallas as pl
from jax.experimental.pallas import tpu as pltpu

K = 2048
M_SHARD = 1024
M_BLK = 256
F = 8192
DY_CHUNK = 512
N_DY = F // DY_CHUNK
CW = int(os.environ.get("KCW", "512"))
NC = F // CW
SUB = CW // DY_CHUNK
H = CW // 2

DO_Y = "y" not in os.environ.get("KSKIP", "")
DO_X = "x" not in os.environ.get("KSKIP", "")
DO_Z = "z" not in os.environ.get("KSKIP", "")
DO_GEMM = not os.environ.get("KNOGEMM")
DO_ADD = not os.environ.get("KNOADD")
DO_FWD = DO_X and DO_Z and not os.environ.get("KNOFWD")


def kernel(x, dy):
    my_x = lax.axis_index("x")
    my_y = lax.axis_index("y")
    my_z = lax.axis_index("z")
    q = 2 * my_z + my_x

    col_me = my_y * M_SHARD + q * M_BLK
    col_pr = (1 - my_y) * M_SHARD + q * M_BLK
    a_me = lax.dynamic_slice(x, (0, col_me), (K, M_BLK)).T
    a_pr = lax.dynamic_slice(x, (0, col_pr), (K, M_BLK)).T

    def body(a_me_ref, a_pr_ref, dy_ref, out_ref,
             dy_vmem, send_buf, recv_buf, red_buf,
             dy_sems, store_sem, y_ss, y_rs, x1_ss, x1_rs, z1_ss, z1_rs,
             z2_ss, z2_rs, x2_ss, x2_rs):
        my_x = lax.axis_index("x")
        my_y = lax.axis_index("y")
        my_z = lax.axis_index("z")
        row0 = (2 * my_z + my_x) * M_BLK
        row0x = (2 * my_z + (1 - my_x)) * M_BLK
        row0z = (2 * (1 - my_z) + my_x) * M_BLK

        y_dev = (my_x, 1 - my_y, my_z)
        x_dev = (1 - my_x, my_y, my_z)
        z_dev = (my_x, my_y, 1 - my_z)

        barrier = pltpu.get_barrier_semaphore()
        for dev in (y_dev, x_dev, z_dev):
            pl.semaphore_signal(barrier, inc=1, device_id=dev,
                                device_id_type=pl.DeviceIdType.MESH)
        pl.semaphore_wait(barrier, 3)

        def exch(src_ref, dst_rows, col_off, width, ss, rs, dev):
            return pltpu.make_async_remote_copy(
                src_ref=src_ref,
                dst_ref=out_ref.at[pl.ds(dst_rows, M_BLK),
                                   pl.ds(col_off, width)],
                send_sem=ss, recv_sem=rs,
                device_id=dev, device_id_type=pl.DeviceIdType.MESH)

        def make_y(c):
            cols = pl.ds(c * CW, CW)
            return pltpu.make_async_remote_copy(
                src_ref=send_buf.at[:, cols], dst_ref=recv_buf.at[:, cols],
                send_sem=y_ss.at[c], recv_sem=y_rs.at[c],
                device_id=y_dev, device_id_type=pl.DeviceIdType.MESH)

        y_rdmas = [make_y(c) for c in range(NC)]
        x1_rdmas = [exch(red_buf.at[:, pl.ds(c * CW, CW)], row0, c * CW, CW,
                         x1_ss.at[c], x1_rs.at[c], x_dev) for c in range(NC)]
        z1_rdmas = [exch(red_buf.at[:, pl.ds(c * CW, CW)], row0, c * CW, CW,
                         z1_ss.at[c], z1_rs.at[c], z_dev) for c in range(NC)]
        z2_rdmas = [exch(out_ref.at[pl.ds(row0x, M_BLK), pl.ds(c * CW, H)],
                         row0x, c * CW, H,
                         z2_ss.at[c], z2_rs.at[c], z_dev) for c in range(NC)]
        x2_rdmas = [exch(out_ref.at[pl.ds(row0z, M_BLK), pl.ds(c * CW + H, H)],
                         row0z, c * CW + H, H,
                         x2_ss.at[c], x2_rs.at[c], x_dev) for c in range(NC)]

        def dy_copy(i):
            cols = pl.ds(i * DY_CHUNK, DY_CHUNK)
            return pltpu.make_async_copy(
                dy_ref.at[:, cols], dy_vmem.at[i % 2], dy_sems.at[i % 2])

        if DO_GEMM:
            copies = [dy_copy(i) for i in range(N_DY)]
            copies[0].start()
            for i in range(N_DY):
                copies[i].wait()
                cols = pl.ds(i * DY_CHUNK, DY_CHUNK)
                d = dy_vmem[i % 2, :, :]
                send_buf[:, cols] = lax.dot_general(
                    a_pr_ref[:, :], d, (((1,), (0,)), ((), ())),
                    preferred_element_type=jnp.float32)
                if DO_Y and (i + 1) % SUB == 0:
                    y_rdmas[(i + 1) // SUB - 1].start()
                if i + 1 < N_DY:
                    copies[i + 1].start()
                red_buf[:, cols] = lax.dot_general(
                    a_me_ref[:, :], d, (((1,), (0,)), ((), ())),
                    preferred_element_type=jnp.float32)
        elif DO_Y:
            for c in range(NC):
                y_rdmas[c].start()

        for c in range(NC):
            cols = pl.ds(c * CW, CW)
            if DO_Y:
                y_rdmas[c].wait_recv()
                if DO_ADD:
                    red_buf[:, cols] = red_buf[:, cols] + recv_buf[:, cols]
            if DO_Z:
                z1_rdmas[c].start()
            if DO_X:
                x1_rdmas[c].start()

        store = pltpu.make_async_copy(
            red_buf, out_ref.at[pl.ds(row0, M_BLK), :], store_sem)
        store.start()

        if DO_X:
            for c in range(NC):
                x1_rdmas[c].wait_recv()
                if DO_FWD:
                    z2_rdmas[c].start()

        if DO_Z:
            for c in range(NC):
                z1_rdmas[c].wait_recv()
                if DO_FWD:
                    x2_rdmas[c].start()

        store.wait()
        for c in range(NC):
            if DO_FWD:
                z2_rdmas[c].wait_recv()
                x2_rdmas[c].wait_recv()
            if DO_Y:
                y_rdmas[c].wait_send()
            if DO_X:
                x1_rdmas[c].wait_send()
            if DO_Z:
                z1_rdmas[c].wait_send()
            if DO_FWD:
                z2_rdmas[c].wait_send()
                x2_rdmas[c].wait_send()

    dma = pltpu.SemaphoreType.DMA
    return pl.pallas_call(
        body,
        out_shape=jax.ShapeDtypeStruct((M_SHARD, F), jnp.float32),
        in_specs=[
            pl.BlockSpec(memory_space=pltpu.VMEM),
            pl.BlockSpec(memory_space=pltpu.VMEM),
            pl.BlockSpec(memory_space=pl.ANY),
        ],
        out_specs=pl.BlockSpec(memory_space=pl.ANY),
        scratch_shapes=[
            pltpu.VMEM((2, K, DY_CHUNK), jnp.float32),
            pltpu.VMEM((M_BLK, F), jnp.float32),
            pltpu.VMEM((M_BLK, F), jnp.float32),
            pltpu.VMEM((M_BLK, F), jnp.float32),
            dma((2,)),
            dma,
            dma((NC,)), dma((NC,)),
            dma((NC,)), dma((NC,)),
            dma((NC,)), dma((NC,)),
            dma((NC,)), dma((NC,)),
            dma((NC,)), dma((NC,)),
        ],
        compiler_params=pltpu.CompilerParams(
            collective_id=0, vmem_limit_bytes=64 * 1024 * 1024),
    )(a_me, a_pr, dy)


# device time: 235134 ns/iter; 1.0505x vs baseline; 1.0069x over previous
import os

import jax
import jax.numpy as jnp
from jax import lax
from jax.experimental import pallas as pl
from jax.experimental.pallas import tpu as pltpu

K = 2048
M_SHARD = 1024
M_BLK = 256
F = 8192
DY_CHUNK = 512
N_DY = F // DY_CHUNK
CW = int(os.environ.get("KCW", "512"))
NC = F // CW
SUB = CW // DY_CHUNK
H = CW // 2

DO_Y = "y" not in os.environ.get("KSKIP", "")
DO_X = "x" not in os.environ.get("KSKIP", "")
DO_Z = "z" not in os.environ.get("KSKIP", "")
DO_GEMM = not os.environ.get("KNOGEMM")
DO_ADD = not os.environ.get("KNOADD")
DO_FWD = DO_X and DO_Z and not os.environ.get("KNOFWD")


def kernel(x, dy):
    my_x = lax.axis_index("x")
    my_y = lax.axis_index("y")
    my_z = lax.axis_index("z")
    q = 2 * my_z + my_x

    col_me = my_y * M_SHARD + q * M_BLK
    col_pr = (1 - my_y) * M_SHARD + q * M_BLK
    a_me = lax.dynamic_slice(x, (0, col_me), (K, M_BLK)).T
    a_pr = lax.dynamic_slice(x, (0, col_pr), (K, M_BLK)).T

    def body(a_me_ref, a_pr_ref, dy_ref, out_ref,
             dy_vmem, send_buf, recv_buf, red_buf,
             dy_sems, store_sem, y_ss, y_rs, x1_ss, x1_rs, z1_ss, z1_rs,
             z2_ss, z2_rs, x2_ss, x2_rs):
        my_x = lax.axis_index("x")
        my_y = lax.axis_index("y")
        my_z = lax.axis_index("z")
        row0 = (2 * my_z + my_x) * M_BLK
        row0x = (2 * my_z + (1 - my_x)) * M_BLK
        row0z = (2 * (1 - my_z) + my_x) * M_BLK

        y_dev = (my_x, 1 - my_y, my_z)
        x_dev = (1 - my_x, my_y, my_z)
        z_dev = (my_x, my_y, 1 - my_z)

        barrier = pltpu.get_barrier_semaphore()
        for dev in (y_dev, x_dev, z_dev):
            pl.semaphore_signal(barrier, inc=1, device_id=dev,
                                device_id_type=pl.DeviceIdType.MESH)
        pl.semaphore_wait(barrier, 3)

        def exch(src_ref, dst_rows, col_off, width, ss, rs, dev):
            return pltpu.make_async_remote_copy(
                src_ref=src_ref,
                dst_ref=out_ref.at[pl.ds(dst_rows, M_BLK),
                                   pl.ds(col_off, width)],
                send_sem=ss, recv_sem=rs,
                device_id=dev, device_id_type=pl.DeviceIdType.MESH)

        def make_y(c):
            cols = pl.ds(c * CW, CW)
            return pltpu.make_async_remote_copy(
                src_ref=send_buf.at[:, cols], dst_ref=recv_buf.at[:, cols],
                send_sem=y_ss.at[c], recv_sem=y_rs.at[c],
                device_id=y_dev, device_id_type=pl.DeviceIdType.MESH)

        y_rdmas = [make_y(c) for c in range(NC)]
        x1_rdmas = [exch(red_buf.at[:, pl.ds(c * CW, CW)], row0, c * CW, CW,
                         x1_ss.at[c], x1_rs.at[c], x_dev) for c in range(NC)]
        z1_rdmas = [exch(red_buf.at[:, pl.ds(c * CW, CW)], row0, c * CW, CW,
                         z1_ss.at[c], z1_rs.at[c], z_dev) for c in range(NC)]
        z2_rdmas = [exch(out_ref.at[pl.ds(row0x, M_BLK), pl.ds(c * CW, H)],
                         row0x, c * CW, H,
                         z2_ss.at[c], z2_rs.at[c], z_dev) for c in range(NC)]
        x2_rdmas = [exch(out_ref.at[pl.ds(row0z, M_BLK), pl.ds(c * CW + H, H)],
                         row0z, c * CW + H, H,
                         x2_ss.at[c], x2_rs.at[c], x_dev) for c in range(NC)]

        def dy_copy(i):
            cols = pl.ds(i * DY_CHUNK, DY_CHUNK)
            return pltpu.make_async_copy(
                dy_ref.at[:, cols], dy_vmem.at[i % 2], dy_sems.at[i % 2])

        if DO_GEMM:
            copies = [dy_copy(i) for i in range(N_DY)]
            copies[0].start()
            for i in range(N_DY):
                copies[i].wait()
                cols = pl.ds(i * DY_CHUNK, DY_CHUNK)
                d = dy_vmem[i % 2, :, :]
                send_buf[:, cols] = lax.dot_general(
                    a_pr_ref[:, :], d, (((1,), (0,)), ((), ())),
                    preferred_element_type=jnp.float32)
                if DO_Y and (i + 1) % SUB == 0:
                    y_rdmas[(i + 1) // SUB - 1].start()
                if i + 1 < N_DY:
                    copies[i + 1].start()
                red_buf[:, cols] = lax.dot_general(
                    a_me_ref[:, :], d, (((1,), (0,)), ((), ())),
                    preferred_element_type=jnp.float32)
        elif DO_Y:
            for c in range(NC):
                y_rdmas[c].start()

        LAG = 2

        def forward(c):
            if DO_X:
                x1_rdmas[c].wait_recv()
                if DO_FWD:
                    z2_rdmas[c].start()
            if DO_Z:
                z1_rdmas[c].wait_recv()
                if DO_FWD:
                    x2_rdmas[c].start()

        for c in range(NC):
            cols = pl.ds(c * CW, CW)
            if DO_Y:
                y_rdmas[c].wait_recv()
                if DO_ADD:
                    red_buf[:, cols] = red_buf[:, cols] + recv_buf[:, cols]
            if DO_Z:
                z1_rdmas[c].start()
            if DO_X:
                x1_rdmas[c].start()
            if c >= LAG:
                forward(c - LAG)

        store = pltpu.make_async_copy(
            red_buf, out_ref.at[pl.ds(row0, M_BLK), :], store_sem)
        store.start()

        for c in range(NC - LAG, NC):
            forward(c)

        store.wait()
        for c in range(NC):
            if DO_FWD:
                z2_rdmas[c].wait_recv()
                x2_rdmas[c].wait_recv()
            if DO_Y:
                y_rdmas[c].wait_send()
            if DO_X:
                x1_rdmas[c].wait_send()
            if DO_Z:
                z1_rdmas[c].wait_send()
            if DO_FWD:
                z2_rdmas[c].wait_send()
                x2_rdmas[c].wait_send()

    dma = pltpu.SemaphoreType.DMA
    return pl.pallas_call(
        body,
        out_shape=jax.ShapeDtypeStruct((M_SHARD, F), jnp.float32),
        in_specs=[
            pl.BlockSpec(memory_space=pltpu.VMEM),
            pl.BlockSpec(memory_space=pltpu.VMEM),
            pl.BlockSpec(memory_space=pl.ANY),
        ],
        out_specs=pl.BlockSpec(memory_space=pl.ANY),
        scratch_shapes=[
            pltpu.VMEM((2, K, DY_CHUNK), jnp.float32),
            pltpu.VMEM((M_BLK, F), jnp.float32),
            pltpu.VMEM((M_BLK, F), jnp.float32),
            pltpu.VMEM((M_BLK, F), jnp.float32),
            dma((2,)),
            dma,
            dma((NC,)), dma((NC,)),
            dma((NC,)), dma((NC,)),
            dma((NC,)), dma((NC,)),
            dma((NC,)), dma((NC,)),
            dma((NC,)), dma((NC,)),
        ],
        compiler_params=pltpu.CompilerParams(
            collective_id=0, vmem_limit_bytes=64 * 1024 * 1024),
    )(a_me, a_pr, dy)
